# baseline (device time: 212354 ns/iter reference)
import jax
import jax.numpy as jnp
from jax import lax
from jax.experimental import pallas as pl
from jax.experimental.pallas import tpu as pltpu

N_DEV = 16
N_IDX = 1024
V_PER = 4096
D = 512


def kernel(table, idx):
    assert table.shape == (V_PER, D), table.shape
    assert idx.shape == (N_IDX,), idx.shape
    idx2 = idx.reshape(N_IDX, 1)

    def body(table_ref, idx_ref, out_ref, comm_ref, send_sems, recv_sems):
        my = lax.axis_index("i")
        left = lax.rem(my - 1 + N_DEV, N_DEV)
        right = lax.rem(my + 1, N_DEV)

        local_idx = idx_ref[:, :] - my * V_PER
        vocab_iota = lax.broadcasted_iota(jnp.int32, (N_IDX, V_PER), 1)
        onehot = (local_idx == vocab_iota).astype(jnp.bfloat16)
        acc = jnp.dot(
            onehot,
            table_ref[:, :].astype(jnp.bfloat16),
            preferred_element_type=jnp.float32,
        )
        out_ref[:, :] = acc
        comm_ref[0, :, :] = acc.astype(jnp.bfloat16)

        barrier_sem = pltpu.get_barrier_semaphore()
        for nbr in [left, right]:
            pl.semaphore_signal(
                barrier_sem, inc=1,
                device_id=(nbr,), device_id_type=pl.DeviceIdType.MESH,
            )
        pl.semaphore_wait(barrier_sem, 2)

        for h in range(N_DEV - 1):
            rdma = pltpu.make_async_remote_copy(
                src_ref=comm_ref.at[h],
                dst_ref=comm_ref.at[h + 1],
                send_sem=send_sems.at[h],
                recv_sem=recv_sems.at[h],
                device_id=(right,),
                device_id_type=pl.DeviceIdType.MESH,
            )
            rdma.start()
            rdma.wait()
            out_ref[:, :] = out_ref[:, :] + comm_ref[h + 1, :, :].astype(
                jnp.float32
            )

    return pl.pallas_call(
        body,
        out_shape=jax.ShapeDtypeStruct((N_IDX, D), jnp.float32),
        in_specs=[
            pl.BlockSpec(memory_space=pltpu.VMEM),
            pl.BlockSpec(memory_space=pltpu.VMEM),
        ],
        out_specs=pl.BlockSpec(memory_space=pltpu.VMEM),
        scratch_shapes=[
            pltpu.VMEM((N_DEV, N_IDX, D), jnp.bfloat16),
            pltpu.SemaphoreType.DMA((N_DEV - 1,)),
            pltpu.SemaphoreType.DMA((N_DEV - 1,)),
        ],
        compiler_params=pltpu.CompilerParams(collective_id=0),
    )(table, idx2)


# device time: 37239 ns/iter; 5.7025x vs baseline; 5.7025x over previous
import jax
import jax.numpy as jnp
from jax import lax
from jax.experimental import pallas as pl
from jax.experimental.pallas import tpu as pltpu

N_DEV = 16
N_IDX = 1024
V_PER = 4096
D = 512
CH = N_IDX // N_DEV


def kernel(table, idx):
    assert table.shape == (V_PER, D), table.shape
    assert idx.shape == (N_IDX,), idx.shape
    idx2 = idx.reshape(N_IDX, 1)

    def body(
        table_ref,
        idx_ref,
        out_ref,
        part_ref,
        land_ref,
        s1_send,
        s1_recv,
        s2_send,
        s2_recv,
    ):
        my = lax.axis_index("i")

        local_idx = idx_ref[:, :] - my * V_PER
        vocab_iota = lax.broadcasted_iota(jnp.int32, (N_IDX, V_PER), 1)
        onehot = (local_idx == vocab_iota).astype(jnp.bfloat16)
        part_ref[:, :] = jnp.dot(
            onehot,
            table_ref[:, :].astype(jnp.bfloat16),
            preferred_element_type=jnp.float32,
        ).astype(jnp.bfloat16)

        barrier_sem = pltpu.get_barrier_semaphore()
        for k in range(1, N_DEV):
            peer = lax.rem(my + k, N_DEV)
            pl.semaphore_signal(
                barrier_sem, inc=1,
                device_id=(peer,), device_id_type=pl.DeviceIdType.MESH,
            )
        pl.semaphore_wait(barrier_sem, N_DEV - 1)

        phase1 = []
        for k in range(1, N_DEV):
            tgt = lax.rem(my + k, N_DEV)
            rdma = pltpu.make_async_remote_copy(
                src_ref=part_ref.at[pl.ds(tgt * CH, CH), :],
                dst_ref=land_ref.at[my],
                send_sem=s1_send.at[k - 1],
                recv_sem=s1_recv,
                device_id=(tgt,),
                device_id_type=pl.DeviceIdType.MESH,
            )
            rdma.start()
            phase1.append(rdma)
        land_ref[my] = part_ref[pl.ds(my * CH, CH), :]
        for rdma in phase1:
            rdma.wait_recv()

        out_ref[pl.ds(my * CH, CH), :] = jnp.sum(
            land_ref[:, :, :].astype(jnp.float32), axis=0
        ).astype(jnp.bfloat16)

        phase2 = []
        for k in range(1, N_DEV):
            tgt = lax.rem(my + k, N_DEV)
            rdma = pltpu.make_async_remote_copy(
                src_ref=out_ref.at[pl.ds(my * CH, CH), :],
                dst_ref=out_ref.at[pl.ds(my * CH, CH), :],
                send_sem=s2_send.at[k - 1],
                recv_sem=s2_recv,
                device_id=(tgt,),
                device_id_type=pl.DeviceIdType.MESH,
            )
            rdma.start()
            phase2.append(rdma)
        for rdma in phase2:
            rdma.wait_recv()
        for rdma in phase1:
            rdma.wait_send()
        for rdma in phase2:
            rdma.wait_send()

    return pl.pallas_call(
        body,
        out_shape=jax.ShapeDtypeStruct((N_IDX, D), jnp.bfloat16),
        in_specs=[
            pl.BlockSpec(memory_space=pltpu.VMEM),
            pl.BlockSpec(memory_space=pltpu.VMEM),
        ],
        out_specs=pl.BlockSpec(memory_space=pltpu.VMEM),
        scratch_shapes=[
            pltpu.VMEM((N_IDX, D), jnp.bfloat16),
            pltpu.VMEM((N_DEV, CH, D), jnp.bfloat16),
            pltpu.SemaphoreType.DMA((N_DEV - 1,)),
            pltpu.SemaphoreType.DMA,
            pltpu.SemaphoreType.DMA((N_DEV - 1,)),
            pltpu.SemaphoreType.DMA,
        ],
        compiler_params=pltpu.CompilerParams(collective_id=0),
    )(table, idx2)
